# baseline (speedup 1.0000x reference)
import functools

import jax
import jax.numpy as jnp
from jax import lax
from jax.experimental import pallas as pl
from jax.experimental.pallas import tpu as pltpu
from jax.experimental.pallas import tpu_sc as plsc

S = 2048
D = 1024
E = 16
F = 1024
C = 128
EC = E * C
RB = 256
NR = S // RB
NC = 2
NS = 16
NW = NC * NS
TPB = S // NW


def _router_body(x_ref, wg_ref, slot_ref, keep_ref, gate_ref, laux_ref,
                 cnt_ref, zrep_ref, carry_ref, acc_ref):
    i = pl.program_id(0)

    @pl.when(i == 0)
    def _():
        carry_ref[...] = jnp.zeros((1, E), jnp.float32)
        acc_ref[...] = jnp.zeros((1, E), jnp.float32)

    x = x_ref[...]
    wg = wg_ref[...]
    logits = jnp.dot(x, wg, preferred_element_type=jnp.float32)
    mx = jnp.max(logits, axis=1, keepdims=True)
    p = jnp.exp(logits - mx)
    gates = p / jnp.sum(p, axis=1, keepdims=True)
    gmax = jnp.max(gates, axis=1, keepdims=True)
    ie = lax.broadcasted_iota(jnp.int32, (RB, E), 1)
    idx1 = jnp.min(jnp.where(gates == gmax, ie, E), axis=1, keepdims=True)
    oh = (ie == idx1).astype(jnp.float32)

    carry = carry_ref[...]
    tri = (lax.broadcasted_iota(jnp.int32, (RB, RB), 0) >=
           lax.broadcasted_iota(jnp.int32, (RB, RB), 1)).astype(jnp.float32)
    incl = jnp.dot(tri, oh, preferred_element_type=jnp.float32) + carry
    pos = incl - 1.0
    pos_s = jnp.sum(pos * oh, axis=1, keepdims=True)
    kept = pos_s < C
    slot_ref[...] = jnp.where(kept, idx1 * C + pos_s.astype(jnp.int32), 0)
    keep_ref[...] = jnp.where(kept, 1, 0)
    gate_ref[...] = jnp.where(kept, gmax, 0.0)
    counts = carry + jnp.sum(oh, axis=0, keepdims=True)
    carry_ref[...] = counts
    me = acc_ref[...] + jnp.sum(gates, axis=0, keepdims=True)
    acc_ref[...] = me

    cnt_post = jnp.minimum(counts, C)
    cnt_ref[...] = cnt_post.astype(jnp.int32)
    laux_ref[...] = jnp.sum(me * counts, axis=1, keepdims=True) * (E / (S * S))
    ie_row = lax.broadcasted_iota(jnp.int32, (1, E), 1)
    space = cnt_post < C
    ffs = jnp.min(jnp.where(space, ie_row, E), axis=1, keepdims=True)
    cnt_at = jnp.sum(jnp.where(ie_row == ffs, cnt_post, 0.0), axis=1,
                     keepdims=True).astype(jnp.int32)
    z = jnp.where(ffs < E, ffs * C + cnt_at, 0)
    zrep_ref[...] = jnp.broadcast_to(z, (1, E))


_router = pl.pallas_call(
    _router_body,
    grid=(NR,),
    in_specs=[
        pl.BlockSpec((RB, D), lambda i: (i, 0)),
        pl.BlockSpec((D, E), lambda i: (0, 0)),
    ],
    out_specs=[
        pl.BlockSpec((RB, 1), lambda i: (i, 0)),
        pl.BlockSpec((RB, 1), lambda i: (i, 0)),
        pl.BlockSpec((RB, 1), lambda i: (i, 0)),
        pl.BlockSpec((1, 1), lambda i: (0, 0)),
        pl.BlockSpec((1, E), lambda i: (0, 0)),
        pl.BlockSpec((1, E), lambda i: (0, 0)),
    ],
    out_shape=[
        jax.ShapeDtypeStruct((S, 1), jnp.int32),
        jax.ShapeDtypeStruct((S, 1), jnp.int32),
        jax.ShapeDtypeStruct((S, 1), jnp.float32),
        jax.ShapeDtypeStruct((1, 1), jnp.float32),
        jax.ShapeDtypeStruct((1, E), jnp.int32),
        jax.ShapeDtypeStruct((1, E), jnp.int32),
    ],
    scratch_shapes=[
        pltpu.VMEM((1, E), jnp.float32),
        pltpu.VMEM((1, E), jnp.float32),
    ],
)


def _dispatch_body(x_hbm, slot_hbm, keep_hbm, gate_hbm,
                   xd_hbm, gps_hbm,
                   aslot_v, akeep_v, agate_v, tfs_v, gps_v, rows_v, sem):
    wid = lax.axis_index("s") * NC + lax.axis_index("c")
    base = wid * TPB

    pltpu.sync_copy(slot_hbm, aslot_v)
    pltpu.sync_copy(keep_hbm, akeep_v)
    pltpu.sync_copy(gate_hbm, agate_v)

    def init_body(j, _):
        tfs_v[pl.ds(j * 16, 16)] = jnp.zeros((16,), jnp.int32)
        gps_v[pl.ds(j * 16, 16)] = jnp.zeros((16,), jnp.float32)
        return 0

    lax.fori_loop(0, TPB // 16, init_body, 0)

    def scat_body(j, _):
        sl = aslot_v[pl.ds(j * 16, 16)] - base
        kp = akeep_v[pl.ds(j * 16, 16)]
        gt = agate_v[pl.ds(j * 16, 16)]
        tok = lax.iota(jnp.int32, 16) + j * 16
        m = (kp > 0) & (sl >= 0) & (sl < TPB)
        slc = jnp.minimum(jnp.maximum(sl, 0), TPB - 1)
        plsc.store_scatter(tfs_v, [slc], tok, mask=m)
        plsc.store_scatter(gps_v, [slc], gt, mask=m)
        return 0

    lax.fori_loop(0, S // 16, scat_body, 0)
    pltpu.sync_copy(gps_v, gps_hbm.at[pl.ds(base, TPB)])
    pltpu.async_copy(x_hbm.at[tfs_v], rows_v, sem).wait()
    pltpu.sync_copy(rows_v, xd_hbm.at[pl.ds(base, TPB)])


def _combine_body(ys_hbm, slot_hbm, keep_hbm, zrep_hbm, out_hbm,
                  slot_v, keep_v, sg_v, z_v, rows_v, sem):
    wid = lax.axis_index("s") * NC + lax.axis_index("c")
    base = wid * TPB

    pltpu.sync_copy(zrep_hbm, z_v)
    z = z_v[...]
    pltpu.sync_copy(slot_hbm.at[pl.ds(base, TPB)], slot_v)
    pltpu.sync_copy(keep_hbm.at[pl.ds(base, TPB)], keep_v)

    def sg_body(j, _):
        sl = slot_v[pl.ds(j * 16, 16)]
        kp = keep_v[pl.ds(j * 16, 16)]
        sg_v[pl.ds(j * 16, 16)] = jnp.where(kp > 0, sl, z)
        return 0

    lax.fori_loop(0, TPB // 16, sg_body, 0)
    pltpu.async_copy(ys_hbm.at[sg_v], rows_v, sem).wait()
    pltpu.sync_copy(rows_v, out_hbm.at[pl.ds(base, TPB)])


@functools.cache
def _sc_kernels():
    mesh = plsc.VectorSubcoreMesh(core_axis_name="c", subcore_axis_name="s",
                                  num_cores=NC, num_subcores=NS)
    params = pltpu.CompilerParams(needs_layout_passes=False)
    dispatch = pl.kernel(
        _dispatch_body,
        out_type=[
            jax.ShapeDtypeStruct((EC, D), jnp.float32),
            jax.ShapeDtypeStruct((EC,), jnp.float32),
        ],
        mesh=mesh,
        compiler_params=params,
        scratch_types=[
            pltpu.VMEM((S,), jnp.int32),
            pltpu.VMEM((S,), jnp.int32),
            pltpu.VMEM((S,), jnp.float32),
            pltpu.VMEM((TPB,), jnp.int32),
            pltpu.VMEM((TPB,), jnp.float32),
            pltpu.VMEM((TPB, D), jnp.float32),
            pltpu.SemaphoreType.DMA,
        ],
    )
    combine = pl.kernel(
        _combine_body,
        out_type=jax.ShapeDtypeStruct((S, D), jnp.float32),
        mesh=mesh,
        compiler_params=params,
        scratch_types=[
            pltpu.VMEM((TPB,), jnp.int32),
            pltpu.VMEM((TPB,), jnp.int32),
            pltpu.VMEM((TPB,), jnp.int32),
            pltpu.VMEM((16,), jnp.int32),
            pltpu.VMEM((TPB, D), jnp.float32),
            pltpu.SemaphoreType.DMA,
        ],
    )
    return dispatch, combine


def _mlp_body(xd_ref, w1_ref, b1_ref, w2_ref, b2_ref, gps_ref, out_ref):
    xb = xd_ref[0]
    h = jnp.dot(xb, w1_ref[0], preferred_element_type=jnp.float32) + b1_ref[0]
    h = jax.nn.gelu(h)
    y = jnp.dot(h, w2_ref[0], preferred_element_type=jnp.float32) + b2_ref[0]
    out_ref[0] = y * gps_ref[0]


_mlp = pl.pallas_call(
    _mlp_body,
    grid=(E,),
    in_specs=[
        pl.BlockSpec((1, C, D), lambda e: (e, 0, 0)),
        pl.BlockSpec((1, D, F), lambda e: (e, 0, 0)),
        pl.BlockSpec((1, 1, F), lambda e: (e, 0, 0)),
        pl.BlockSpec((1, F, D), lambda e: (e, 0, 0)),
        pl.BlockSpec((1, 1, D), lambda e: (e, 0, 0)),
        pl.BlockSpec((1, C, 1), lambda e: (e, 0, 0)),
    ],
    out_specs=pl.BlockSpec((1, C, D), lambda e: (e, 0, 0)),
    out_shape=jax.ShapeDtypeStruct((E, C, D), jnp.float32),
)


def kernel(hidden_states, wg, w1, b1, w2, b2):
    x = hidden_states.reshape(S, D)
    slot2, keep2, gate2, laux, cnt2, zrep2 = _router(x, wg)
    slot = slot2.reshape(S)
    keep = keep2.reshape(S)
    gate = gate2.reshape(S)
    cnt = cnt2.reshape(E)
    _dispatch, _combine = _sc_kernels()
    xd, gps = _dispatch(x, slot, keep, gate)
    ys = _mlp(xd.reshape(E, C, D), w1, b1.reshape(E, 1, F), w2,
              b2.reshape(E, 1, D), gps.reshape(E, C, 1))
    out = _combine(ys.reshape(EC, D), slot, keep, zrep2.reshape(E))
    return out.reshape(hidden_states.shape), laux.reshape(()), cnt
